# baseline (speedup 1.0000x reference)
import functools

import jax
import jax.numpy as jnp
from jax import lax
from jax.experimental import pallas as pl
from jax.experimental.pallas import tpu as pltpu
from jax.experimental.pallas import tpu_sc as plsc

B, C = 16384, 100
NC, NS = 2, 16
NW = NC * NS
BPW = B // NW
SLAB = BPW * C
LANES = 16
CHUNK = 128
NCHUNK = BPW // CHUNK
WPAD = 128

NB = 8
VR = 128


@functools.cache
def _make_sc_gather():
    return functools.partial(
        pl.kernel,
        out_type=[
            jax.ShapeDtypeStruct((VR, B // VR), jnp.float32),
            jax.ShapeDtypeStruct((VR, B // VR), jnp.float32),
        ],
        mesh=plsc.VectorSubcoreMesh(core_axis_name="c", subcore_axis_name="s"),
        scratch_types=[
            pltpu.VMEM_SHARED((NS * SLAB + WPAD,), jnp.float32),
            pltpu.VMEM((BPW,), jnp.int32),
            pltpu.VMEM((NCHUNK, CHUNK), jnp.int32),
            pltpu.VMEM((NCHUNK, CHUNK), jnp.int32),
            pltpu.VMEM((NCHUNK, CHUNK), jnp.float32),
            pltpu.VMEM((NCHUNK, CHUNK), jnp.float32),
            pltpu.SemaphoreType.DMA,
            pltpu.SemaphoreType.DMA,
        ],
    )(_sc_gather_body)


def _sc_gather_body(oflat, tgt, wpad, vals, wgt,
                    shr_v, t_v, idx_v, tix_v, val_v, wg_v, sem, sem2):
    cid = lax.axis_index("c")
    sid = lax.axis_index("s")
    wid = sid * NC + cid
    base = wid * BPW
    lbase = sid * SLAB
    slab_cp = pltpu.async_copy(
        oflat.at[pl.ds(base * C, SLAB)], shr_v.at[pl.ds(lbase, SLAB)], sem2
    )
    w_cp = pltpu.async_copy(wpad, shr_v.at[pl.ds(NS * SLAB, WPAD)], sem2)
    pltpu.sync_copy(tgt.at[pl.ds(base, BPW)], t_v)
    iota = lax.iota(jnp.int32, LANES)
    for j in range(BPW // LANES):
        t16 = t_v[pl.ds(j * LANES, LANES)]
        loc = (j * LANES) + iota
        ch, col = divmod(j * LANES, CHUNK)
        idx_v[ch, pl.ds(col, LANES)] = lbase + loc * C + t16
        tix_v[ch, pl.ds(col, LANES)] = NS * SLAB + t16
    slab_cp.wait()
    w_cp.wait()
    copies = []
    for ch in range(NCHUNK):
        copies.append(pltpu.async_copy(
            shr_v.at[idx_v.at[ch]], val_v.at[ch], sem
        ))
        copies.append(pltpu.async_copy(
            shr_v.at[tix_v.at[ch]], wg_v.at[ch], sem
        ))
    for cp in copies:
        cp.wait()
    pltpu.sync_copy(val_v, vals.at[pl.ds(wid * NCHUNK, NCHUNK), :])
    pltpu.sync_copy(wg_v, wgt.at[pl.ds(wid * NCHUNK, NCHUNK), :])


def _tc_dense_body(o_ref, w_ref, out_ref, acc_ref):
    i = pl.program_id(0)
    x = o_ref[...]
    part = jnp.sum(jnp.maximum(jnp.log1p(-x), -100.0) * w_ref[...])

    @pl.when(i == 0)
    def _():
        acc_ref[0, 0] = 0.0

    acc_ref[0, 0] += part

    @pl.when(i == NB - 1)
    def _():
        out_ref[0, 0] = acc_ref[0, 0]


def _tc_dense(o2d, w2d):
    return pl.pallas_call(
        _tc_dense_body,
        grid=(NB,),
        in_specs=[
            pl.BlockSpec((B // NB, C), lambda i: (i, 0)),
            pl.BlockSpec((1, C), lambda i: (0, 0)),
        ],
        out_specs=pl.BlockSpec(memory_space=pltpu.SMEM),
        out_shape=jax.ShapeDtypeStruct((1, 1), jnp.float32),
        scratch_shapes=[pltpu.SMEM((1, 1), jnp.float32)],
    )(o2d, w2d)


def _tc_combine_body(d_ref, v_ref, g_ref, out_ref):
    v = v_ref[...]
    corr = jnp.sum(
        g_ref[...]
        * (jnp.maximum(jnp.log(v), -100.0) - jnp.maximum(jnp.log1p(-v), -100.0))
    )
    out_ref[0, 0] = (d_ref[0, 0] + corr) * (-1.0 / (B * C))


def _tc_combine(dense, v2d, g2d):
    return pl.pallas_call(
        _tc_combine_body,
        in_specs=[
            pl.BlockSpec(memory_space=pltpu.SMEM),
            pl.BlockSpec((VR, B // VR), lambda: (0, 0)),
            pl.BlockSpec((VR, B // VR), lambda: (0, 0)),
        ],
        out_specs=pl.BlockSpec(memory_space=pltpu.SMEM),
        out_shape=jax.ShapeDtypeStruct((1, 1), jnp.float32),
    )(dense, v2d, g2d)


def kernel(output, target, weight):
    oflat = output.reshape(B * C)
    wpad = jnp.pad(weight, (0, WPAD - C))
    tgt = target.reshape(B)
    vals2d, wg2d = _make_sc_gather()(oflat, tgt, wpad)
    dense = _tc_dense(output, weight.reshape(1, C))
    out = _tc_combine(dense, vals2d, wg2d)
    return out[0, 0]


# speedup vs baseline: 1.0454x; 1.0454x over previous
import functools

import jax
import jax.numpy as jnp
from jax import lax
from jax.experimental import pallas as pl
from jax.experimental.pallas import tpu as pltpu
from jax.experimental.pallas import tpu_sc as plsc

B, C = 16384, 100
NC, NS = 2, 16
NW = NC * NS
BPW = B // NW
SLAB = BPW * C
LANES = 16
CHUNK = 128
NCHUNK = BPW // CHUNK
WPAD = 128

NB = 2
VR = 128


@functools.cache
def _make_sc_gather():
    return functools.partial(
        pl.kernel,
        out_type=[
            jax.ShapeDtypeStruct((VR, B // VR), jnp.float32),
            jax.ShapeDtypeStruct((VR, B // VR), jnp.float32),
        ],
        mesh=plsc.VectorSubcoreMesh(core_axis_name="c", subcore_axis_name="s"),
        scratch_types=[
            pltpu.VMEM_SHARED((NS * SLAB + WPAD,), jnp.float32),
            pltpu.VMEM((BPW,), jnp.int32),
            pltpu.VMEM((NCHUNK, CHUNK), jnp.int32),
            pltpu.VMEM((NCHUNK, CHUNK), jnp.int32),
            pltpu.VMEM((NCHUNK, CHUNK), jnp.float32),
            pltpu.VMEM((NCHUNK, CHUNK), jnp.float32),
            pltpu.SemaphoreType.DMA,
            pltpu.SemaphoreType.DMA,
        ],
    )(_sc_gather_body)


def _sc_gather_body(oflat, tgt, wpad, vals, wgt,
                    shr_v, t_v, idx_v, tix_v, val_v, wg_v, sem, sem2):
    cid = lax.axis_index("c")
    sid = lax.axis_index("s")
    wid = sid * NC + cid
    base = wid * BPW
    lbase = sid * SLAB
    slab_cp = pltpu.async_copy(
        oflat.at[pl.ds(base * C, SLAB)], shr_v.at[pl.ds(lbase, SLAB)], sem2
    )
    w_cp = pltpu.async_copy(wpad, shr_v.at[pl.ds(NS * SLAB, WPAD)], sem2)
    pltpu.sync_copy(tgt.at[pl.ds(base, BPW)], t_v)
    iota = lax.iota(jnp.int32, LANES)
    for j in range(BPW // LANES):
        t16 = t_v[pl.ds(j * LANES, LANES)]
        loc = (j * LANES) + iota
        ch, col = divmod(j * LANES, CHUNK)
        idx_v[ch, pl.ds(col, LANES)] = lbase + loc * C + t16
        tix_v[ch, pl.ds(col, LANES)] = NS * SLAB + t16
    slab_cp.wait()
    w_cp.wait()
    copies = []
    for ch in range(NCHUNK):
        copies.append(pltpu.async_copy(
            shr_v.at[idx_v.at[ch]], val_v.at[ch], sem
        ))
        copies.append(pltpu.async_copy(
            shr_v.at[tix_v.at[ch]], wg_v.at[ch], sem
        ))
    for cp in copies:
        cp.wait()
    pltpu.sync_copy(val_v, vals.at[pl.ds(wid * NCHUNK, NCHUNK), :])
    pltpu.sync_copy(wg_v, wgt.at[pl.ds(wid * NCHUNK, NCHUNK), :])


def _tc_dense_body(o_ref, w_ref, out_ref, acc_ref):
    i = pl.program_id(0)
    x = o_ref[...]
    part = jnp.sum(jnp.maximum(jnp.log1p(-x), -100.0) * w_ref[...])

    @pl.when(i == 0)
    def _():
        acc_ref[0, 0] = 0.0

    acc_ref[0, 0] += part

    @pl.when(i == NB - 1)
    def _():
        out_ref[0, 0] = acc_ref[0, 0]


def _tc_dense(o2d, w2d):
    return pl.pallas_call(
        _tc_dense_body,
        grid=(NB,),
        in_specs=[
            pl.BlockSpec((B // NB, C), lambda i: (i, 0)),
            pl.BlockSpec((1, C), lambda i: (0, 0)),
        ],
        out_specs=pl.BlockSpec(memory_space=pltpu.SMEM),
        out_shape=jax.ShapeDtypeStruct((1, 1), jnp.float32),
        scratch_shapes=[pltpu.SMEM((1, 1), jnp.float32)],
    )(o2d, w2d)


def _tc_combine_body(d_ref, v_ref, g_ref, out_ref):
    v = v_ref[...]
    corr = jnp.sum(
        g_ref[...]
        * (jnp.maximum(jnp.log(v), -100.0) - jnp.maximum(jnp.log1p(-v), -100.0))
    )
    out_ref[0, 0] = (d_ref[0, 0] + corr) * (-1.0 / (B * C))


def _tc_combine(dense, v2d, g2d):
    return pl.pallas_call(
        _tc_combine_body,
        in_specs=[
            pl.BlockSpec(memory_space=pltpu.SMEM),
            pl.BlockSpec((VR, B // VR), lambda: (0, 0)),
            pl.BlockSpec((VR, B // VR), lambda: (0, 0)),
        ],
        out_specs=pl.BlockSpec(memory_space=pltpu.SMEM),
        out_shape=jax.ShapeDtypeStruct((1, 1), jnp.float32),
    )(dense, v2d, g2d)


def kernel(output, target, weight):
    oflat = output.reshape(B * C)
    wpad = jnp.pad(weight, (0, WPAD - C))
    tgt = target.reshape(B)
    vals2d, wg2d = _make_sc_gather()(oflat, tgt, wpad)
    dense = _tc_dense(output, weight.reshape(1, C))
    out = _tc_combine(dense, vals2d, wg2d)
    return out[0, 0]
